# Initial kernel scaffold; baseline (speedup 1.0000x reference)
import jax
import jax.numpy as jnp
from jax.experimental import pallas as pl


def kernel(x, edge_index, batch, W1, b1, W2, b2, gamma, beta, Wd, bd, gnoise):
    raise NotImplementedError("write your pallas kernel here")


# speedup vs baseline: 2.5250x; 2.5250x over previous
import functools

import jax
import jax.numpy as jnp
from jax import lax
from jax.experimental import pallas as pl
from jax.experimental.pallas import tpu as pltpu
from jax.experimental.pallas import tpu_sc as plsc

_N = 10000
_D = 128
_B = 64
_L = 8
_E = 320000
_NC = 10

_K = 128
_CHUNKS = 80
_TILES = 32
_EPAD = _TILES * _CHUNKS * _K
_RPT = 624
_TAIL = _N - 16 * _RPT
_ACC_ROWS = _N + 16


def _segsum_body(h, zeros, srcp, dstp, out,
                 src0, src1, dst0, dst1, rows0, rows1,
                 isem0, isem1, gsem0, gsem1, acc):
    cid = lax.axis_index("c")
    sid = lax.axis_index("s")
    gid = cid * 16 + sid
    cbase = gid * _CHUNKS

    row0 = sid * _RPT

    @pl.when(cid == 0)
    def _():
        pltpu.sync_copy(h.at[pl.ds(row0, _RPT)], acc.at[pl.ds(row0, _RPT)])

    @pl.when(cid != 0)
    def _():
        pltpu.sync_copy(zeros.at[pl.ds(row0, _RPT)], acc.at[pl.ds(row0, _RPT)])

    @pl.when(jnp.logical_and(sid == 0, cid == 0))
    def _():
        pltpu.sync_copy(h.at[pl.ds(16 * _RPT, _TAIL)],
                        acc.at[pl.ds(16 * _RPT, _TAIL)])

    @pl.when(jnp.logical_and(sid == 0, cid != 0))
    def _():
        pltpu.sync_copy(zeros.at[pl.ds(16 * _RPT, _TAIL)],
                        acc.at[pl.ds(16 * _RPT, _TAIL)])

    @pl.when(sid == 15)
    def _():
        pltpu.sync_copy(zeros.at[pl.ds(0, 16)], acc.at[pl.ds(_N, 16)])

    plsc.subcore_barrier()

    def _idxload(j, sbuf, dbuf, isem):
        pltpu.async_copy(srcp.at[cbase + j], sbuf, isem)
        pltpu.async_copy(dstp.at[cbase + j], dbuf, isem)

    def _idxwait(sbuf, dbuf, isem):
        pltpu.make_async_copy(srcp.at[0], sbuf, isem).wait()
        pltpu.make_async_copy(dstp.at[0], dbuf, isem).wait()

    def _gather(sbuf, rows, gsem):
        pltpu.async_copy(h.at[sbuf], rows, gsem)

    def _gatherwait(rows, gsem):
        pltpu.make_async_copy(h.at[src0], rows, gsem).wait()

    def _scatter(rows, dbuf):
        pltpu.sync_copy(rows, acc.at[dbuf], add=True)

    _idxload(0, src0, dst0, isem0)
    _idxload(1, src1, dst1, isem1)
    _idxwait(src0, dst0, isem0)
    _gather(src0, rows0, gsem0)
    _idxwait(src1, dst1, isem1)
    _gather(src1, rows1, gsem1)

    def _step(i, carry):
        j = 2 * i
        _gatherwait(rows0, gsem0)
        _scatter(rows0, dst0)
        _idxload(j + 2, src0, dst0, isem0)
        _gatherwait(rows1, gsem1)
        _scatter(rows1, dst1)
        _idxload(j + 3, src1, dst1, isem1)
        _idxwait(src0, dst0, isem0)
        _gather(src0, rows0, gsem0)
        _idxwait(src1, dst1, isem1)
        _gather(src1, rows1, gsem1)
        return carry

    lax.fori_loop(0, (_CHUNKS - 2) // 2, _step, 0)
    _gatherwait(rows0, gsem0)
    _scatter(rows0, dst0)
    _gatherwait(rows1, gsem1)
    _scatter(rows1, dst1)

    plsc.subcore_barrier()
    pltpu.sync_copy(acc.at[pl.ds(row0, _RPT)], out.at[cid].at[pl.ds(row0, _RPT)])

    @pl.when(sid == 0)
    def _():
        pltpu.sync_copy(acc.at[pl.ds(16 * _RPT, _TAIL)],
                        out.at[cid].at[pl.ds(16 * _RPT, _TAIL)])


@functools.cache
def _make_segsum():
    return pl.kernel(
        _segsum_body,
        out_type=jax.ShapeDtypeStruct((2, _N, _D), jnp.float32),
        mesh=plsc.VectorSubcoreMesh(core_axis_name="c", subcore_axis_name="s"),
        scratch_types=[
            pltpu.VMEM((_K,), jnp.int32),
            pltpu.VMEM((_K,), jnp.int32),
            pltpu.VMEM((_K,), jnp.int32),
            pltpu.VMEM((_K,), jnp.int32),
            pltpu.VMEM((_K, _D), jnp.float32),
            pltpu.VMEM((_K, _D), jnp.float32),
            pltpu.SemaphoreType.DMA,
            pltpu.SemaphoreType.DMA,
            pltpu.SemaphoreType.DMA,
            pltpu.SemaphoreType.DMA,
            pltpu.VMEM_SHARED((_ACC_ROWS, _D), jnp.float32),
        ],
    )


def _dense_core(p_ref, gn_ref, batch_ref, w1_ref, b1_ref, w2_ref, b2_ref,
                gam_ref, bet_ref):
    z0 = p_ref[0] + p_ref[1]
    t = jnp.dot(z0.astype(jnp.bfloat16), w1_ref[...].astype(jnp.bfloat16),
                preferred_element_type=jnp.float32)
    t = t + b1_ref[...]
    t = jnp.where(t >= 0, t, 0.01 * t)
    z = jnp.dot(t.astype(jnp.bfloat16), w2_ref[...].astype(jnp.bfloat16),
                preferred_element_type=jnp.float32)
    z = z + b2_ref[...]
    mu = jnp.mean(z, axis=0, keepdims=True)
    d = z - mu
    var = jnp.mean(d * d, axis=0, keepdims=True)
    zn = gam_ref[...] * d / jnp.sqrt(var + 1e-5) + bet_ref[...]
    g = zn + gn_ref[...]
    m = jnp.max(g, axis=1, keepdims=True)
    lanes = lax.broadcasted_iota(jnp.int32, (_N, _D), 1)
    first = jnp.min(jnp.where(g == m, lanes, _D), axis=1, keepdims=True)
    nc = (lanes == first).astype(jnp.float32)
    h = jnp.where(zn >= 0, zn, 0.01 * zn)
    rows = lax.broadcasted_iota(jnp.int32, (_B, _N), 0)
    bo = (rows == batch_ref[...]).astype(jnp.float32)
    cnt = jnp.dot(bo, nc, preferred_element_type=jnp.float32)
    gc = (cnt > 0.5).astype(jnp.float32)
    return nc, h, gc


def _layer_body(p_ref, gn_ref, batch_ref, w1_ref, b1_ref, w2_ref, b2_ref,
                gam_ref, bet_ref, nc_ref, h_ref, gc_ref):
    nc, h, gc = _dense_core(p_ref, gn_ref, batch_ref, w1_ref, b1_ref, w2_ref,
                            b2_ref, gam_ref, bet_ref)
    nc_ref[...] = nc
    h_ref[...] = h
    gc_ref[...] = gc


def _last_layer_body(p_ref, gn_ref, batch_ref, w1_ref, b1_ref, w2_ref, b2_ref,
                     gam_ref, bet_ref, wd_ref, bd_ref, nc_ref, h_ref, gc_ref,
                     out_ref):
    nc, h, gc = _dense_core(p_ref, gn_ref, batch_ref, w1_ref, b1_ref, w2_ref,
                            b2_ref, gam_ref, bet_ref)
    nc_ref[...] = nc
    h_ref[...] = h
    gc_ref[...] = gc
    out_ref[...] = jnp.dot(gc.astype(jnp.bfloat16),
                           wd_ref[...].astype(jnp.bfloat16),
                           preferred_element_type=jnp.float32) + bd_ref[...]


_layer = pl.pallas_call(
    _layer_body,
    out_shape=(
        jax.ShapeDtypeStruct((_N, _D), jnp.float32),
        jax.ShapeDtypeStruct((_N, _D), jnp.float32),
        jax.ShapeDtypeStruct((_B, _D), jnp.float32),
    ),
)

_last_layer = pl.pallas_call(
    _last_layer_body,
    out_shape=(
        jax.ShapeDtypeStruct((_N, _D), jnp.float32),
        jax.ShapeDtypeStruct((_N, _D), jnp.float32),
        jax.ShapeDtypeStruct((_B, _D), jnp.float32),
        jax.ShapeDtypeStruct((_B, _D), jnp.float32),
    ),
)


def kernel(x, edge_index, batch, W1, b1, W2, b2, gamma, beta, Wd, bd, gnoise):
    src = edge_index[0]
    dst = edge_index[1]
    pad = _EPAD - _E
    srcp = jnp.concatenate([src, jnp.zeros((pad,), jnp.int32)]).reshape(
        _TILES * _CHUNKS, _K)
    dstp = jnp.concatenate([dst, jnp.full((pad,), _N, jnp.int32)]).reshape(
        _TILES * _CHUNKS, _K)
    zeros = jnp.zeros((_N, _D), jnp.float32)
    batch2d = batch.reshape(1, _N)
    wd_pad = jnp.zeros((_D, _D), jnp.float32).at[:, :_NC].set(Wd)
    bd_pad = jnp.zeros((1, _D), jnp.float32).at[0, :_NC].set(bd)

    segsum = _make_segsum()
    h = x
    ncs = []
    gcs = []
    out = None
    for i in range(_L):
        p = segsum(h, zeros, srcp, dstp)
        args = (p, gnoise[i], batch2d, W1[i], b1[i].reshape(1, _D), W2[i],
                b2[i].reshape(1, _D), gamma[i].reshape(1, _D),
                beta[i].reshape(1, _D))
        if i < _L - 1:
            nc, h, gc = _layer(*args)
        else:
            nc, h, gc, out = _last_layer(*(args + (wd_pad, bd_pad)))
        ncs.append(nc)
        gcs.append(gc)
    return out[:, :_NC], jnp.stack(ncs), jnp.stack(gcs)


# speedup vs baseline: 2.9657x; 1.1746x over previous
import functools

import jax
import jax.numpy as jnp
from jax import lax
from jax.experimental import pallas as pl
from jax.experimental.pallas import tpu as pltpu
from jax.experimental.pallas import tpu_sc as plsc

_N = 10000
_D = 128
_B = 64
_L = 8
_E = 320000
_NC = 10

_CK = 80
_IDXMAX = 10080
_RPT = 624
_TAIL = _N - 16 * _RPT
_ACC_ROWS = _N + 16
_STG = 16


def _rank_chunks(rank):
    return jnp.where(rank < 11, 126, jnp.where(rank < 15, 123, 122))


def _rank_offset(rank):
    return jnp.where(
        rank < 11, 10080 * rank,
        jnp.where(rank < 15, 110880 + 9840 * (rank - 11), 150240))


def _segsum_body(h, zeros, ssrc, sdst, out,
                 srcv, dstc, rows, stage, idsb, ids16, curbuf, accbuf,
                 gsem0, acc):
    cid = lax.axis_index("c")
    sid = lax.axis_index("s")
    nchunks = _rank_chunks(sid)
    ebase = 160000 * cid + _rank_offset(sid)

    pltpu.sync_copy(ssrc.at[pl.ds(ebase, _IDXMAX)], srcv)

    row0 = sid * _RPT
    pltpu.sync_copy(zeros.at[pl.ds(row0, _RPT)], acc.at[pl.ds(row0, _RPT)])

    @pl.when(sid == 0)
    def _():
        pltpu.sync_copy(zeros.at[pl.ds(16 * _RPT, _TAIL)],
                        acc.at[pl.ds(16 * _RPT, _TAIL)])

    plsc.subcore_barrier()

    iota16 = lax.iota(jnp.int32, 16)
    nfull = jnp.full((16,), _N, jnp.int32)
    zvec = jnp.zeros((16,), jnp.float32)

    curbuf[pl.ds(0, 16)] = nfull
    for k in range(8):
        accbuf[pl.ds(16 * k, 16)] = zvec

    def _chunk(c, carry):
        pltpu.sync_copy(sdst.at[pl.ds(ebase + c * _CK, _CK)], dstc)
        pltpu.async_copy(h.at[srcv.at[pl.ds(c * _CK, _CK)]], rows,
                         gsem0).wait()
        curv = curbuf[pl.ds(0, 16)]
        accs = tuple(accbuf[pl.ds(16 * k, 16)] for k in range(8))
        idv = nfull
        for e in range(_CK):
            dnb = dstc[e, pl.ds(0, 16)]
            chv = dnb != curv
            srow = stage.at[e]
            for k in range(8):
                srow[pl.ds(16 * k, 16)] = accs[k]
            idv = jnp.where(iota16 == (e % 16),
                            jnp.where(chv, curv, nfull), idv)
            if e % 16 == 15:
                idsb[pl.ds(e - 15, 16)] = idv
                idv = nfull
            new = tuple(rows[e, pl.ds(16 * k, 16)] for k in range(8))
            accs = tuple(
                jnp.where(chv, new[k], accs[k] + new[k]) for k in range(8))
            curv = dnb
        curbuf[pl.ds(0, 16)] = curv
        for k in range(8):
            accbuf[pl.ds(16 * k, 16)] = accs[k]
        pltpu.sync_copy(stage, acc.at[idsb], add=True)
        return carry

    lax.fori_loop(0, nchunks, _chunk, 0)

    curv = curbuf[pl.ds(0, 16)]
    srow = stage.at[0]
    for k in range(8):
        srow[pl.ds(16 * k, 16)] = accbuf[pl.ds(16 * k, 16)]
    ids16[pl.ds(0, 16)] = jnp.where(iota16 == 0, curv, nfull)
    pltpu.sync_copy(stage.at[pl.ds(0, 16)], acc.at[ids16], add=True)

    plsc.subcore_barrier()
    pltpu.sync_copy(acc.at[pl.ds(row0, _RPT)], out.at[cid].at[pl.ds(row0, _RPT)])

    @pl.when(sid == 0)
    def _():
        pltpu.sync_copy(acc.at[pl.ds(16 * _RPT, _TAIL)],
                        out.at[cid].at[pl.ds(16 * _RPT, _TAIL)])


@functools.cache
def _make_segsum():
    return pl.kernel(
        _segsum_body,
        out_type=jax.ShapeDtypeStruct((2, _N, _D), jnp.float32),
        mesh=plsc.VectorSubcoreMesh(core_axis_name="c", subcore_axis_name="s"),
        scratch_types=[
            pltpu.VMEM((_IDXMAX,), jnp.int32),
            pltpu.VMEM((_CK, 16), jnp.int32),
            pltpu.VMEM((_CK, _D), jnp.float32),
            pltpu.VMEM((_CK, _D), jnp.float32),
            pltpu.VMEM((_CK,), jnp.int32),
            pltpu.VMEM((16,), jnp.int32),
            pltpu.VMEM((16,), jnp.int32),
            pltpu.VMEM((_D,), jnp.float32),
            pltpu.SemaphoreType.DMA,
            pltpu.VMEM_SHARED((_ACC_ROWS, _D), jnp.float32),
        ],
    )


def _mlp_body(h_ref, p_ref, w1_ref, b1_ref, w2_ref, b2_ref, z_ref):
    z0 = h_ref[...] + (p_ref[0] + p_ref[1])
    t = jnp.dot(z0.astype(jnp.bfloat16), w1_ref[...].astype(jnp.bfloat16),
                preferred_element_type=jnp.float32)
    t = t + b1_ref[...]
    t = jnp.where(t >= 0, t, 0.01 * t)
    z = jnp.dot(t.astype(jnp.bfloat16), w2_ref[...].astype(jnp.bfloat16),
                preferred_element_type=jnp.float32)
    z_ref[...] = z + b2_ref[...]


_mlp = pl.pallas_call(
    _mlp_body,
    out_shape=jax.ShapeDtypeStruct((_N, _D), jnp.float32),
)


def _post_core(z_ref, mu_ref, var_ref, gn_ref, batch_ref, gam_ref, bet_ref):
    z = z_ref[...]
    zn = gam_ref[...] * (z - mu_ref[...]) / jnp.sqrt(var_ref[...] + 1e-5) \
        + bet_ref[...]
    g = zn + gn_ref[...]
    m = jnp.max(g, axis=1, keepdims=True)
    lanes = lax.broadcasted_iota(jnp.int32, (_N, _D), 1)
    first = jnp.min(jnp.where(g == m, lanes, _D), axis=1, keepdims=True)
    nc = (lanes == first).astype(jnp.float32)
    h = jnp.where(zn >= 0, zn, 0.01 * zn)
    rows = lax.broadcasted_iota(jnp.int32, (_B, _N), 0)
    bo = (rows == batch_ref[...]).astype(jnp.float32)
    cnt = jnp.dot(bo, nc, preferred_element_type=jnp.float32)
    gc = (cnt > 0.5).astype(jnp.float32)
    return nc, h, gc


def _post_body(z_ref, mu_ref, var_ref, gn_ref, batch_ref, gam_ref, bet_ref,
               nc_ref, h_ref, gc_ref):
    nc, h, gc = _post_core(z_ref, mu_ref, var_ref, gn_ref, batch_ref,
                           gam_ref, bet_ref)
    nc_ref[...] = nc
    h_ref[...] = h
    gc_ref[...] = gc


def _post_last_body(z_ref, mu_ref, var_ref, gn_ref, batch_ref, gam_ref,
                    bet_ref, wd_ref, bd_ref, nc_ref, h_ref, gc_ref, out_ref):
    nc, h, gc = _post_core(z_ref, mu_ref, var_ref, gn_ref, batch_ref,
                           gam_ref, bet_ref)
    nc_ref[...] = nc
    h_ref[...] = h
    gc_ref[...] = gc
    out_ref[...] = jnp.dot(gc.astype(jnp.bfloat16),
                           wd_ref[...].astype(jnp.bfloat16),
                           preferred_element_type=jnp.float32) + bd_ref[...]


_post = pl.pallas_call(
    _post_body,
    out_shape=(
        jax.ShapeDtypeStruct((_N, _D), jnp.float32),
        jax.ShapeDtypeStruct((_N, _D), jnp.float32),
        jax.ShapeDtypeStruct((_B, _D), jnp.float32),
    ),
)

_post_last = pl.pallas_call(
    _post_last_body,
    out_shape=(
        jax.ShapeDtypeStruct((_N, _D), jnp.float32),
        jax.ShapeDtypeStruct((_N, _D), jnp.float32),
        jax.ShapeDtypeStruct((_B, _D), jnp.float32),
        jax.ShapeDtypeStruct((_B, _D), jnp.float32),
    ),
)


def kernel(x, edge_index, batch, W1, b1, W2, b2, gamma, beta, Wd, bd, gnoise):
    src = edge_index[0]
    dst = edge_index[1]
    perm = jnp.argsort(dst)
    pad = jnp.zeros((_IDXMAX * 32 - _E,), jnp.int32)
    ssrc = jnp.concatenate([src[perm], pad])
    sdst1 = jnp.concatenate([dst[perm], pad])
    sdst = jnp.broadcast_to(sdst1[:, None], (sdst1.shape[0], 16))
    zeros = jnp.zeros((_N, _D), jnp.float32)
    batch2d = batch.reshape(1, _N)
    wd_pad = jnp.zeros((_D, _D), jnp.float32).at[:, :_NC].set(Wd)
    bd_pad = jnp.zeros((1, _D), jnp.float32).at[0, :_NC].set(bd)

    segsum = _make_segsum()
    h = x
    ncs = []
    gcs = []
    out = None
    for i in range(_L):
        p = segsum(h, zeros, ssrc, sdst)
        z = _mlp(h, p, W1[i], b1[i].reshape(1, _D), W2[i],
                 b2[i].reshape(1, _D))
        mu = jnp.mean(z, axis=0, keepdims=True)
        var = jnp.var(z, axis=0, keepdims=True)
        args = (z, mu, var, gnoise[i], batch2d, gamma[i].reshape(1, _D),
                beta[i].reshape(1, _D))
        if i < _L - 1:
            nc, h, gc = _post(*args)
        else:
            nc, h, gc, out = _post_last(*(args + (wd_pad, bd_pad)))
        ncs.append(nc)
        gcs.append(gc)
    return out[:, :_NC], jnp.stack(ncs), jnp.stack(gcs)
